# baseline (speedup 1.0000x reference)
import functools
import math

import jax
import jax.numpy as jnp
from jax.experimental import pallas as pl
from jax.experimental.pallas import tpu as pltpu


def _round_up(v, m):
    return ((v + m - 1) // m) * m


def _fused_kernel(jsc_ref, i_ref, rbf_ref, ev_ref, x_ref, vecf_ref,
                  w1_ref, b1_ref, w2_ref, b2_ref, we_ref, be_ref, cs_ref,
                  outx_ref, outv_ref, tab8_ref, tile_ref, wes_ref, bes_ref,
                  acc_ref, *, hidden, te):
    H = hidden
    n_nodes = x_ref.shape[0]
    n_chunks = 3 * H // 128
    S = te + 1

    @pl.when(pl.program_id(0) == 0)
    def _init():
        h = jnp.dot(x_ref[...], w1_ref[...],
                    preferred_element_type=jnp.float32) + b1_ref[...]
        h = h * jax.nn.sigmoid(h) * (1.0 / 0.6)
        xh = jnp.dot(h, w2_ref[...],
                     preferred_element_type=jnp.float32) + b2_ref[...]
        for c in range(n_chunks):
            tab8_ref[c:c + 8 * n_nodes:8, :] = xh[:, c * 128:(c + 1) * 128]
            tab8_ref[n_chunks + c:n_chunks + c + 8 * n_nodes:8, :] = \
                vecf_ref[:, c, :]
        wes_ref[...] = we_ref[...] * cs_ref[...]
        bes_ref[...] = be_ref[...] * cs_ref[...]
        acc_ref[...] = jnp.zeros_like(acc_ref)

    base = pl.program_id(0) * te

    for mi in range(te):
        jj = pl.multiple_of(jsc_ref[base + mi], 8)
        tile_ref[mi:mi + 8 * S:S, :] = tab8_ref[pl.ds(jj, 8), :]

    chunks = [tile_ref[pl.ds(c * S, te), :] for c in range(2 * n_chunks)]

    ii = i_ref[0]
    scat_t = (jax.lax.broadcasted_iota(jnp.int32, (n_nodes, te), 0) == ii
              ).astype(jnp.float32)

    rbf_h = jnp.dot(rbf_ref[...], wes_ref[...],
                    preferred_element_type=jnp.float32) + bes_ref[...]

    x1 = chunks[0] * rbf_h[:, :H]
    x2 = chunks[1] * rbf_h[:, H:2 * H]
    x3 = chunks[2] * rbf_h[:, 2 * H:]

    ev = ev_ref[...]
    pieces = [x3]
    for k in range(3):
        vk = x1 * chunks[3 + k] + x2 * ev[:, k:k + 1]
        pieces.append(vk)
    msgs = jnp.concatenate(pieces, axis=-1)

    acc_ref[...] += jnp.dot(scat_t, msgs, preferred_element_type=jnp.float32)

    @pl.when(pl.program_id(0) == pl.num_programs(0) - 1)
    def _finalize():
        outx_ref[...] = acc_ref[:, :H]
        for k in range(3):
            outv_ref[:, k, :] = acc_ref[:, (k + 1) * H:(k + 2) * H]


@functools.partial(jax.jit, static_argnames=("tile_e",))
def _message_passing(x, vec, edge_index, edge_rbf, edge_vector,
                     we_t, be, w1_t, b1, w2_t, b2, *, tile_e=1024):
    N, H = x.shape
    E, F = edge_rbf.shape

    te = min(tile_e, _round_up(E, 8))
    e_pad = _round_up(E, te)
    pe = e_pad - E
    eidx = edge_index.astype(jnp.int32)
    rbf = edge_rbf
    ev = edge_vector
    if pe:
        eidx = jnp.concatenate([
            jnp.pad(eidx[:1], ((0, 0), (0, pe))),
            jnp.pad(eidx[1:], ((0, 0), (0, pe)), constant_values=N)], axis=0)
        rbf = jnp.pad(rbf, ((0, pe), (0, 0)))
        ev = jnp.pad(ev, ((0, pe), (0, 0)))
    j8 = eidx[0] * 8
    i3 = eidx[1].reshape(e_pad // te, 1, te)

    inv3 = 1.0 / math.sqrt(3.0)
    invh = 1.0 / math.sqrt(float(H))
    col_scale = jnp.concatenate([
        jnp.full((2 * H,), inv3 * invh, jnp.float32),
        jnp.full((H,), inv3, jnp.float32)]).reshape(1, -1)

    S = te + 1
    tile_rows = _round_up((te - 1) + 7 * S + 1, 8)

    kern = functools.partial(_fused_kernel, hidden=H, te=te)
    d_x, d_vec = pl.pallas_call(
        kern,
        out_shape=[jax.ShapeDtypeStruct((N, H), jnp.float32),
                   jax.ShapeDtypeStruct((N, 3, H), jnp.float32)],
        grid_spec=pltpu.PrefetchScalarGridSpec(
            num_scalar_prefetch=1,
            grid=(e_pad // te,),
            in_specs=[
                pl.BlockSpec((1, 1, te), lambda s, jr: (s, 0, 0)),
                pl.BlockSpec((te, F), lambda s, jr: (s, 0)),
                pl.BlockSpec((te, 3), lambda s, jr: (s, 0)),
                pl.BlockSpec((N, H), lambda s, jr: (0, 0)),
                pl.BlockSpec((N, 3, H), lambda s, jr: (0, 0, 0)),
                pl.BlockSpec((H, H // 2), lambda s, jr: (0, 0)),
                pl.BlockSpec((1, H // 2), lambda s, jr: (0, 0)),
                pl.BlockSpec((H // 2, 3 * H), lambda s, jr: (0, 0)),
                pl.BlockSpec((1, 3 * H), lambda s, jr: (0, 0)),
                pl.BlockSpec((F, 3 * H), lambda s, jr: (0, 0)),
                pl.BlockSpec((1, 3 * H), lambda s, jr: (0, 0)),
                pl.BlockSpec((1, 3 * H), lambda s, jr: (0, 0)),
            ],
            out_specs=[pl.BlockSpec((N, H), lambda s, jr: (0, 0)),
                       pl.BlockSpec((N, 3, H), lambda s, jr: (0, 0, 0))],
            scratch_shapes=[
                pltpu.VMEM((8 * N, 128), jnp.float32),
                pltpu.VMEM((tile_rows, 128), jnp.float32),
                pltpu.VMEM((F, 3 * H), jnp.float32),
                pltpu.VMEM((1, 3 * H), jnp.float32),
                pltpu.VMEM((N, 4 * H), jnp.float32),
            ],
        ),
        compiler_params=pltpu.CompilerParams(
            dimension_semantics=("arbitrary",),
            vmem_limit_bytes=64 * 1024 * 1024),
    )(j8, i3, rbf, ev, x, vec,
      w1_t, b1.reshape(1, -1), w2_t, b2.reshape(1, -1), we_t,
      be.reshape(1, -1), col_scale)

    return d_x, d_vec


def kernel(x, vec, edge_index, edge_rbf, edge_vector, we_t, be, w1_t, b1, w2_t, b2):
    return _message_passing(x, vec, edge_index, edge_rbf, edge_vector,
                            we_t, be, w1_t, b1, w2_t, b2)


# speedup vs baseline: 1.0321x; 1.0321x over previous
import functools
import math

import jax
import jax.numpy as jnp
from jax.experimental import pallas as pl
from jax.experimental.pallas import tpu as pltpu


def _round_up(v, m):
    return ((v + m - 1) // m) * m


def _fused_kernel(jsc_ref, i_ref, rbf_ref, ev_ref, x_ref, vecf_ref,
                  w1_ref, b1_ref, w2_ref, b2_ref, we_ref, be_ref, cs_ref,
                  outx_ref, outv_ref, tab8_ref, tile_ref, wes_ref, bes_ref,
                  acc_ref, *, hidden, te):
    H = hidden
    n_nodes = x_ref.shape[0]
    n_chunks = 3 * H // 128
    S = te + 1

    @pl.when(pl.program_id(0) == 0)
    def _init():
        h = jnp.dot(x_ref[...], w1_ref[...],
                    preferred_element_type=jnp.float32) + b1_ref[...]
        h = h * jax.nn.sigmoid(h) * (1.0 / 0.6)
        xh = jnp.dot(h, w2_ref[...],
                     preferred_element_type=jnp.float32) + b2_ref[...]
        for c in range(n_chunks):
            tab8_ref[c:c + 8 * n_nodes:8, :] = xh[:, c * 128:(c + 1) * 128]
            tab8_ref[n_chunks + c:n_chunks + c + 8 * n_nodes:8, :] = \
                vecf_ref[:, c * 128:(c + 1) * 128]
        wes_ref[...] = we_ref[...] * cs_ref[...]
        bes_ref[...] = be_ref[...] * cs_ref[...]
        acc_ref[...] = jnp.zeros_like(acc_ref)

    base = pl.program_id(0) * te

    for mi in range(te):
        jj = pl.multiple_of(jsc_ref[base + mi], 8)
        tile_ref[mi:mi + 8 * S:S, :] = tab8_ref[pl.ds(jj, 8), :]

    chunks = [tile_ref[pl.ds(c * S, te), :] for c in range(2 * n_chunks)]

    ii = i_ref[0]
    scat_t = (jax.lax.broadcasted_iota(jnp.int32, (n_nodes, te), 0) == ii
              ).astype(jnp.float32)

    rbf_h = jnp.dot(rbf_ref[...], wes_ref[...],
                    preferred_element_type=jnp.float32) + bes_ref[...]

    x1 = chunks[0] * rbf_h[:, :H]
    x2 = chunks[1] * rbf_h[:, H:2 * H]
    x3 = chunks[2] * rbf_h[:, 2 * H:]

    ev = ev_ref[...]
    pieces = [x3]
    for k in range(3):
        vk = x1 * chunks[3 + k] + x2 * ev[:, k:k + 1]
        pieces.append(vk)
    msgs = jnp.concatenate(pieces, axis=-1)

    acc_ref[...] += jnp.dot(scat_t, msgs, preferred_element_type=jnp.float32)

    @pl.when(pl.program_id(0) == pl.num_programs(0) - 1)
    def _finalize():
        outx_ref[...] = acc_ref[:, :H]
        for k in range(3):
            outv_ref[:, k, :] = acc_ref[:, (k + 1) * H:(k + 2) * H]


@functools.partial(jax.jit, static_argnames=("tile_e",))
def _message_passing(x, vec, edge_index, edge_rbf, edge_vector,
                     we_t, be, w1_t, b1, w2_t, b2, *, tile_e=1024):
    N, H = x.shape
    E, F = edge_rbf.shape

    te = min(tile_e, _round_up(E, 8))
    e_pad = _round_up(E, te)
    pe = e_pad - E
    eidx = edge_index.astype(jnp.int32)
    rbf = edge_rbf
    ev = edge_vector
    if pe:
        eidx = jnp.concatenate([
            jnp.pad(eidx[:1], ((0, 0), (0, pe))),
            jnp.pad(eidx[1:], ((0, 0), (0, pe)), constant_values=N)], axis=0)
        rbf = jnp.pad(rbf, ((0, pe), (0, 0)))
        ev = jnp.pad(ev, ((0, pe), (0, 0)))
    j8 = eidx[0] * 8
    i3 = eidx[1].reshape(e_pad // te, 1, te)
    vec_flat = vec.reshape(N, 3 * H)

    inv3 = 1.0 / math.sqrt(3.0)
    invh = 1.0 / math.sqrt(float(H))
    col_scale = jnp.concatenate([
        jnp.full((2 * H,), inv3 * invh, jnp.float32),
        jnp.full((H,), inv3, jnp.float32)]).reshape(1, -1)

    S = te + 1
    tile_rows = _round_up((te - 1) + 7 * S + 1, 8)

    kern = functools.partial(_fused_kernel, hidden=H, te=te)
    d_x, d_vec = pl.pallas_call(
        kern,
        out_shape=[jax.ShapeDtypeStruct((N, H), jnp.float32),
                   jax.ShapeDtypeStruct((N, 3, H), jnp.float32)],
        grid_spec=pltpu.PrefetchScalarGridSpec(
            num_scalar_prefetch=1,
            grid=(e_pad // te,),
            in_specs=[
                pl.BlockSpec((1, 1, te), lambda s, jr: (s, 0, 0)),
                pl.BlockSpec((te, F), lambda s, jr: (s, 0)),
                pl.BlockSpec((te, 3), lambda s, jr: (s, 0)),
                pl.BlockSpec((N, H), lambda s, jr: (0, 0)),
                pl.BlockSpec((N, 3 * H), lambda s, jr: (0, 0)),
                pl.BlockSpec((H, H // 2), lambda s, jr: (0, 0)),
                pl.BlockSpec((1, H // 2), lambda s, jr: (0, 0)),
                pl.BlockSpec((H // 2, 3 * H), lambda s, jr: (0, 0)),
                pl.BlockSpec((1, 3 * H), lambda s, jr: (0, 0)),
                pl.BlockSpec((F, 3 * H), lambda s, jr: (0, 0)),
                pl.BlockSpec((1, 3 * H), lambda s, jr: (0, 0)),
                pl.BlockSpec((1, 3 * H), lambda s, jr: (0, 0)),
            ],
            out_specs=[pl.BlockSpec((N, H), lambda s, jr: (0, 0)),
                       pl.BlockSpec((N, 3, H), lambda s, jr: (0, 0, 0))],
            scratch_shapes=[
                pltpu.VMEM((8 * N, 128), jnp.float32),
                pltpu.VMEM((tile_rows, 128), jnp.float32),
                pltpu.VMEM((F, 3 * H), jnp.float32),
                pltpu.VMEM((1, 3 * H), jnp.float32),
                pltpu.VMEM((N, 4 * H), jnp.float32),
            ],
        ),
        compiler_params=pltpu.CompilerParams(
            dimension_semantics=("arbitrary",),
            vmem_limit_bytes=64 * 1024 * 1024),
    )(j8, i3, rbf, ev, x, vec_flat,
      w1_t, b1.reshape(1, -1), w2_t, b2.reshape(1, -1), we_t,
      be.reshape(1, -1), col_scale)

    return d_x, d_vec


def kernel(x, vec, edge_index, edge_rbf, edge_vector, we_t, be, w1_t, b1, w2_t, b2):
    return _message_passing(x, vec, edge_index, edge_rbf, edge_vector,
                            we_t, be, w1_t, b1, w2_t, b2)
